# baseline (speedup 1.0000x reference)
import functools

import jax
import jax.numpy as jnp
from jax import lax
from jax.experimental import pallas as pl
from jax.experimental.pallas import tpu as pltpu
from jax.experimental.pallas import tpu_sc as plsc

_PERCENT_T = 0.02
_N = 512 * 512
_K = int(round(_N * _PERCENT_T))
_NROWS = 64
_NTILES = 32
_ROWS_PER_TILE = _NROWS // _NTILES
_CH = 32768
_NCHUNK = _N // _CH
_L12_BINS = 2048
_L3_BINS = 1024
_HIST_WORDS = _L12_BINS * 16
_UNROLL = 8


def _monotone_key(x):
    b = lax.bitcast_convert_type(x, jnp.int32)
    return jnp.where(b < 0, b ^ jnp.int32(0x7FFFFFFF), b)


def _decode_key(key):
    fb = jnp.where(key < 0, key ^ jnp.int32(0x7FFFFFFF), key)
    return lax.bitcast_convert_type(fb, jnp.float32)


def _sc_body(cam_ref, out_ref, hist, vhist, chunks, outv, sem):
    cid = lax.axis_index("c")
    sid = lax.axis_index("s")
    wid = sid * 2 + cid
    lane = lax.iota(jnp.int32, 16)
    ones = jnp.ones((16,), jnp.int32)
    zeros_i = jnp.zeros((16,), jnp.int32)
    zeros_f = jnp.zeros((16,), jnp.float32)

    def zero_ref(ref, nwords, zval):
        def zb(i):
            ref[pl.ds(i * 16, 16)] = zval
        plsc.parallel_loop(0, nwords // 16, unroll=_UNROLL)(zb)

    def stream_pass(row, inner, carry):
        pend = pltpu.async_copy(cam_ref.at[row, 0], chunks[0], sem)
        for ci in range(_NCHUNK):
            pend.wait()
            if ci + 1 < _NCHUNK:
                pend = pltpu.async_copy(
                    cam_ref.at[row, ci + 1], chunks[(ci + 1) % 2], sem)
            carry = inner(chunks[ci % 2], carry)
        return carry

    def group_lane_sums(ref, g):
        acc = ref[pl.ds(g * 256, 16)]
        for v in range(1, 16):
            acc = acc + ref[pl.ds(g * 256 + v * 16, 16)]
        return acc

    def resolve_bin(gc, cum_bg, target):
        cum_b = cum_bg
        h = jnp.int32(-1)
        above = jnp.int32(0)
        for b in reversed(range(16)):
            cnt = jnp.sum(hist[pl.ds((gc * 16 + b) * 16, 16)])
            crosses = (h < 0) & (cum_b + cnt >= target)
            h = jnp.where(crosses, gc * 16 + b, h)
            above = jnp.where(crosses, cum_b, above)
            cum_b = cum_b + cnt
        return h, above

    def find_bin(nbins, target):
        def cond(c):
            _, cum = c
            return cum < target

        def body(c):
            g, cum = c
            return (g - 1, cum + jnp.sum(group_lane_sums(hist, g)))

        g, cum = lax.while_loop(
            cond, body, (jnp.int32(nbins // 16 - 1), jnp.int32(0)))
        gc = g + 1
        cum_bg = cum - jnp.sum(group_lane_sums(hist, gc))
        return resolve_bin(gc, cum_bg, target)

    def run_row(row):
        zero_ref(hist, _HIST_WORDS, zeros_i)

        def p1(buf, carry):
            def b1(ii):
                x = buf[pl.ds(ii * 16, 16)]
                key = _monotone_key(x)
                bin1 = (lax.shift_right_arithmetic(key, 21)
                        + jnp.int32(1024))
                plsc.addupdate_scatter(hist, [bin1 * 16 + lane], ones)
            plsc.parallel_loop(0, _CH // 16, unroll=16)(b1)
            return carry

        stream_pass(row, p1, 0)
        h1, above1 = find_bin(_L12_BINS, jnp.int32(_K))
        h1m = h1 - jnp.int32(1024)

        zero_ref(hist, _HIST_WORDS, zeros_i)

        def p2(buf, carry):
            def b2(ii):
                x = buf[pl.ds(ii * 16, 16)]
                key = _monotone_key(x)
                sel = lax.shift_right_arithmetic(key, 21) == h1m
                bin2 = (lax.shift_right_arithmetic(key, 10)
                        & jnp.int32(0x7FF))
                plsc.addupdate_scatter(hist, [bin2 * 16 + lane], ones,
                                       mask=sel)
            plsc.parallel_loop(0, _CH // 16, unroll=16)(b2)
            return carry

        stream_pass(row, p2, 0)
        h2, _ = find_bin(_L12_BINS, jnp.int32(_K) - above1)
        top22 = (h1m << 11) | h2

        zero_ref(hist, _L3_BINS * 16, zeros_i)
        zero_ref(vhist, _L3_BINS * 16, zeros_f)

        def p3(buf, carry):
            def b3(ii, c):
                sv, cv = c
                x = buf[pl.ds(ii * 16, 16)]
                key = _monotone_key(x)
                hi22 = lax.shift_right_arithmetic(key, 10)
                inbin = hi22 == top22
                above = hi22 > top22
                addr = (key & jnp.int32(0x3FF)) * 16 + lane
                plsc.addupdate_scatter(hist, [addr], ones, mask=inbin)
                plsc.addupdate_scatter(vhist, [addr], x, mask=inbin)
                sv = sv + jnp.where(above, x, zeros_f)
                cv = cv + jnp.where(above, ones, zeros_i)
                return (sv, cv)
            return plsc.parallel_loop(0, _CH // 16, unroll=_UNROLL,
                                      carry=carry)(b3)

        sumv, cntv = stream_pass(row, p3, (zeros_f, zeros_i))
        c_above = jnp.sum(cntv)
        sum_above = jnp.sum(sumv)
        k3 = jnp.int32(_K) - c_above

        def cond3(c):
            _, cum, _ = c
            return cum < k3

        def body3(c):
            g, cum, vacc = c
            vacc = vacc + group_lane_sums(vhist, g)
            return (g - 1, cum + jnp.sum(group_lane_sums(hist, g)), vacc)

        g, cum, vacc = lax.while_loop(
            cond3, body3,
            (jnp.int32(_L3_BINS // 16 - 1), jnp.int32(0), zeros_f))
        gc = g + 1
        cum_bg = cum - jnp.sum(group_lane_sums(hist, gc))
        h3, above3 = resolve_bin(gc, cum_bg, k3)
        vfix = zeros_f
        for b in range(16):
            vec = vhist[pl.ds((gc * 16 + b) * 16, 16)]
            keep = (gc * 16 + b) > h3
            vfix = vfix + jnp.where(keep, vec, zeros_f)
        vl_cross = group_lane_sums(vhist, gc)
        sum_bins_above = jnp.sum(vacc) - jnp.sum(vl_cross) + jnp.sum(vfix)
        ties = (k3 - above3).astype(jnp.float32)
        val_l = _decode_key((top22 << 10) | h3)
        total = sum_above + sum_bins_above + ties * val_l
        return total * jnp.float32(1.0 / _K)

    m0 = run_row(wid * _ROWS_PER_TILE)
    m1 = run_row(wid * _ROWS_PER_TILE + 1)
    outv[...] = jnp.where(lane == 0, m0, jnp.where(lane == 1, m1, zeros_f))
    pltpu.sync_copy(outv, out_ref.at[wid])


@functools.partial(
    pl.kernel,
    mesh=plsc.VectorSubcoreMesh(core_axis_name="c", subcore_axis_name="s"),
    out_type=jax.ShapeDtypeStruct((_NTILES, 16), jnp.float32),
    scratch_types=[
        pltpu.VMEM((_HIST_WORDS,), jnp.int32),
        pltpu.VMEM((_L3_BINS * 16,), jnp.float32),
        pltpu.VMEM((_CH,), jnp.float32),
        pltpu.VMEM((_CH,), jnp.float32),
        pltpu.VMEM((16,), jnp.float32),
        pltpu.SemaphoreType.DMA,
    ],
    compiler_params=pltpu.CompilerParams(needs_layout_passes=False),
)
def _sc_topk(cam_ref, out_ref, hist, vhist, chunk_a, chunk_b, outv, sem):
    _sc_body(cam_ref, out_ref, hist, vhist, (chunk_a, chunk_b), outv, sem)


@jax.jit
def kernel(cam):
    batch, ncls, h, w = cam.shape
    rows = cam.reshape(_NROWS, _NCHUNK, _CH)
    out = _sc_topk(rows)
    return out[:, :_ROWS_PER_TILE].reshape(batch, ncls)


# speedup vs baseline: 1.0768x; 1.0768x over previous
import functools

import jax
import jax.numpy as jnp
from jax import lax
from jax.experimental import pallas as pl
from jax.experimental.pallas import tpu as pltpu
from jax.experimental.pallas import tpu_sc as plsc

_PERCENT_T = 0.02
_N = 512 * 512
_K = int(round(_N * _PERCENT_T))
_NROWS = 64
_NTILES = 32
_ROWS_PER_TILE = _NROWS // _NTILES
_CH = 32768
_NCHUNK = _N // _CH
_L12_BINS = 2048
_L3_BINS = 1024
_HIST_WORDS = _L12_BINS * 16
_UNROLL = 8


def _monotone_key(x):
    b = lax.bitcast_convert_type(x, jnp.int32)
    return jnp.where(b < 0, b ^ jnp.int32(0x7FFFFFFF), b)


def _decode_key(key):
    fb = jnp.where(key < 0, key ^ jnp.int32(0x7FFFFFFF), key)
    return lax.bitcast_convert_type(fb, jnp.float32)


def _sc_body(cam_ref, out_ref, hist, vhist, chunks, outv, sem):
    cid = lax.axis_index("c")
    sid = lax.axis_index("s")
    wid = sid * 2 + cid
    lane = lax.iota(jnp.int32, 16)
    ones = jnp.ones((16,), jnp.int32)
    zeros_i = jnp.zeros((16,), jnp.int32)
    zeros_f = jnp.zeros((16,), jnp.float32)

    def zero_ref(ref, nwords, zval):
        def zb(i):
            ref[pl.ds(i * 16, 16)] = zval
        plsc.parallel_loop(0, nwords // 16, unroll=_UNROLL)(zb)

    def stream_pass(row, inner, carry):
        pend = pltpu.async_copy(cam_ref.at[row, 0], chunks[0], sem)
        for ci in range(_NCHUNK):
            pend.wait()
            if ci + 1 < _NCHUNK:
                pend = pltpu.async_copy(
                    cam_ref.at[row, ci + 1], chunks[(ci + 1) % 2], sem)
            carry = inner(chunks[ci % 2], carry)
        return carry

    def group_lane_sums(ref, g):
        acc = ref[pl.ds(g * 256, 16)]
        for v in range(1, 16):
            acc = acc + ref[pl.ds(g * 256 + v * 16, 16)]
        return acc

    def resolve_bin(gc, cum_bg, target):
        cum_b = cum_bg
        h = jnp.int32(-1)
        above = jnp.int32(0)
        for b in reversed(range(16)):
            cnt = jnp.sum(hist[pl.ds((gc * 16 + b) * 16, 16)])
            crosses = (h < 0) & (cum_b + cnt >= target)
            h = jnp.where(crosses, gc * 16 + b, h)
            above = jnp.where(crosses, cum_b, above)
            cum_b = cum_b + cnt
        return h, above

    def find_bin(nbins, target):
        def cond(c):
            _, cum = c
            return cum < target

        def body(c):
            g, cum = c
            return (g - 1, cum + jnp.sum(group_lane_sums(hist, g)))

        g, cum = lax.while_loop(
            cond, body, (jnp.int32(nbins // 16 - 1), jnp.int32(0)))
        gc = g + 1
        cum_bg = cum - jnp.sum(group_lane_sums(hist, gc))
        return resolve_bin(gc, cum_bg, target)

    def run_row(row):
        zero_ref(hist, _HIST_WORDS, zeros_i)

        def p1(buf, carry):
            def b1(ii):
                x = buf[pl.ds(ii * 16, 16)]
                key = _monotone_key(x)
                bin1 = (lax.shift_right_arithmetic(key, 21)
                        + jnp.int32(1024))
                plsc.addupdate_scatter(hist, [bin1 * 16 + lane], ones)
            plsc.parallel_loop(0, _CH // 16, unroll=_UNROLL)(b1)
            return carry

        stream_pass(row, p1, 0)
        h1, above1 = find_bin(_L12_BINS, jnp.int32(_K))
        h1m = h1 - jnp.int32(1024)

        zero_ref(hist, _HIST_WORDS, zeros_i)

        def p2(buf, carry):
            def b2(ii):
                x = buf[pl.ds(ii * 16, 16)]
                key = _monotone_key(x)
                sel = lax.shift_right_arithmetic(key, 21) == h1m
                bin2 = (lax.shift_right_arithmetic(key, 10)
                        & jnp.int32(0x7FF))
                plsc.addupdate_scatter(hist, [bin2 * 16 + lane], ones,
                                       mask=sel)
            plsc.parallel_loop(0, _CH // 16, unroll=_UNROLL)(b2)
            return carry

        stream_pass(row, p2, 0)
        h2, _ = find_bin(_L12_BINS, jnp.int32(_K) - above1)
        top22 = (h1m << 11) | h2

        zero_ref(hist, _L3_BINS * 16, zeros_i)
        zero_ref(vhist, _L3_BINS * 16, zeros_f)

        def p3(buf, carry):
            def b3(ii, c):
                sv, cv = c
                x = buf[pl.ds(ii * 16, 16)]
                key = _monotone_key(x)
                hi22 = lax.shift_right_arithmetic(key, 10)
                inbin = hi22 == top22
                above = hi22 > top22
                addr = (key & jnp.int32(0x3FF)) * 16 + lane
                plsc.addupdate_scatter(hist, [addr], ones, mask=inbin)
                plsc.addupdate_scatter(vhist, [addr], x, mask=inbin)
                sv = sv + jnp.where(above, x, zeros_f)
                cv = cv + jnp.where(above, ones, zeros_i)
                return (sv, cv)
            return plsc.parallel_loop(0, _CH // 16, unroll=_UNROLL,
                                      carry=carry)(b3)

        sumv, cntv = stream_pass(row, p3, (zeros_f, zeros_i))
        c_above = jnp.sum(cntv)
        sum_above = jnp.sum(sumv)
        k3 = jnp.int32(_K) - c_above

        def cond3(c):
            _, cum, _ = c
            return cum < k3

        def body3(c):
            g, cum, vacc = c
            vacc = vacc + group_lane_sums(vhist, g)
            return (g - 1, cum + jnp.sum(group_lane_sums(hist, g)), vacc)

        g, cum, vacc = lax.while_loop(
            cond3, body3,
            (jnp.int32(_L3_BINS // 16 - 1), jnp.int32(0), zeros_f))
        gc = g + 1
        cum_bg = cum - jnp.sum(group_lane_sums(hist, gc))
        h3, above3 = resolve_bin(gc, cum_bg, k3)
        vfix = zeros_f
        for b in range(16):
            vec = vhist[pl.ds((gc * 16 + b) * 16, 16)]
            keep = (gc * 16 + b) > h3
            vfix = vfix + jnp.where(keep, vec, zeros_f)
        vl_cross = group_lane_sums(vhist, gc)
        sum_bins_above = jnp.sum(vacc) - jnp.sum(vl_cross) + jnp.sum(vfix)
        ties = (k3 - above3).astype(jnp.float32)
        val_l = _decode_key((top22 << 10) | h3)
        total = sum_above + sum_bins_above + ties * val_l
        return total * jnp.float32(1.0 / _K)

    m0 = run_row(wid * _ROWS_PER_TILE)
    m1 = run_row(wid * _ROWS_PER_TILE + 1)
    outv[...] = jnp.where(lane == 0, m0, jnp.where(lane == 1, m1, zeros_f))
    pltpu.sync_copy(outv, out_ref.at[wid])


@functools.partial(
    pl.kernel,
    mesh=plsc.VectorSubcoreMesh(core_axis_name="c", subcore_axis_name="s"),
    out_type=jax.ShapeDtypeStruct((_NTILES, 16), jnp.float32),
    scratch_types=[
        pltpu.VMEM((_HIST_WORDS,), jnp.int32),
        pltpu.VMEM((_L3_BINS * 16,), jnp.float32),
        pltpu.VMEM((_CH,), jnp.float32),
        pltpu.VMEM((_CH,), jnp.float32),
        pltpu.VMEM((16,), jnp.float32),
        pltpu.SemaphoreType.DMA,
    ],
    compiler_params=pltpu.CompilerParams(needs_layout_passes=False),
)
def _sc_topk(cam_ref, out_ref, hist, vhist, chunk_a, chunk_b, outv, sem):
    _sc_body(cam_ref, out_ref, hist, vhist, (chunk_a, chunk_b), outv, sem)


@jax.jit
def kernel(cam):
    batch, ncls, h, w = cam.shape
    rows = cam.reshape(_NROWS, _NCHUNK, _CH)
    out = _sc_topk(rows)
    return out[:, :_ROWS_PER_TILE].reshape(batch, ncls)
